# Initial kernel scaffold; baseline (speedup 1.0000x reference)
import jax
import jax.numpy as jnp
from jax.experimental import pallas as pl


def kernel(idx, targets, token_embedding):
    raise NotImplementedError("write your pallas kernel here")


# speedup vs baseline: 1.3616x; 1.3616x over previous
import functools

import jax
import jax.numpy as jnp
from jax import lax
from jax.experimental import pallas as pl
from jax.experimental.pallas import tpu as pltpu
from jax.experimental.pallas import tpu_sc as plsc

V = 1000
C = 1000
NC, NS = 2, 16
NW = NC * NS
NBUF = 4
DIST = 2


def _rowlse_body(tab_ref, out_ref, copy_ref):
    x = tab_ref[...]
    m = jnp.max(x, axis=1, keepdims=True)
    s = jnp.sum(jnp.exp(x - m), axis=1, keepdims=True)
    out_ref[...] = jnp.broadcast_to(m + jnp.log(s), (V, 128))
    copy_ref[...] = x


def _rowlse(table):
    out, copy = pl.pallas_call(
        _rowlse_body,
        out_shape=[jax.ShapeDtypeStruct((V, 128), jnp.float32),
                   jax.ShapeDtypeStruct((V, C), jnp.float32)],
    )(table)
    return out[:, 0] + jnp.zeros((V,), jnp.float32), copy.reshape(V * C)


def _sc_body(n_tokens, tab, tabflat, idx_hbm, tgt_hbm, rowlse_hbm,
             out_hbm, part_hbm,
             idx_v, tgt_v, fidx_v, pick_v, lse_v, bufs, acc_v,
             psem, lsem, *sems):
    gsems = sems[:NBUF]
    wsems = sems[NBUF:]
    tok = n_tokens // NW
    g_total = tok // 16
    wid = lax.axis_index("s") * NC + lax.axis_index("c")
    base = wid * tok

    pltpu.sync_copy(idx_hbm.at[pl.ds(base, tok)], idx_v)
    pltpu.sync_copy(tgt_hbm.at[pl.ds(base, tok)], tgt_v)

    def fidx_body(g, c):
        sl = pl.ds(pl.multiple_of(g * 16, 16), 16)
        fidx_v[sl] = idx_v[sl] * C + tgt_v[sl]
        return c

    lax.fori_loop(0, g_total, fidx_body, 0)

    n_chunks = (tok + 127) // 128
    def chunk(k):
        size = min(128, tok - k * 128)
        return pl.ds(k * 128, size)

    for k in range(n_chunks):
        pltpu.async_copy(tabflat.at[fidx_v.at[chunk(k)]],
                         pick_v.at[chunk(k)], psem)
        pltpu.async_copy(rowlse_hbm.at[idx_v.at[chunk(k)]],
                         lse_v.at[chunk(k)], lsem)

    def idx16(g):
        return idx_v[pl.ds(pl.multiple_of(g * 16, 16), 16)]

    for s in range(DIST):
        pltpu.async_copy(tab.at[idx16(s)], bufs.at[s], gsems[s])

    def outer(o, c):
        for s in range(NBUF):
            g = o * NBUF + s
            pltpu.make_async_copy(tab.at[idx16(g)], bufs.at[s],
                                  gsems[s]).wait()
            row0 = pl.multiple_of(base + g * 16, 16)
            pltpu.async_copy(bufs.at[s], out_hbm.at[pl.ds(row0, 16)],
                             wsems[s])

            s2 = (s + DIST) % NBUF
            gp = g + DIST - NBUF

            @pl.when(gp >= 0)
            def _wait_prev_write():
                r0 = pl.multiple_of(base + gp * 16, 16)
                pltpu.make_async_copy(bufs.at[s2],
                                      out_hbm.at[pl.ds(r0, 16)],
                                      wsems[s2]).wait()

            @pl.when(g + DIST < g_total)
            def _prefetch():
                pltpu.async_copy(tab.at[idx16(g + DIST)], bufs.at[s2],
                                 gsems[s2])
        return c

    lax.fori_loop(0, g_total // NBUF, outer, 0)

    for g in range(g_total - NBUF + DIST, g_total):
        s = g % NBUF
        row0 = pl.multiple_of(base + g * 16, 16)
        pltpu.make_async_copy(bufs.at[s], out_hbm.at[pl.ds(row0, 16)],
                              wsems[s]).wait()

    for k in range(n_chunks):
        pltpu.make_async_copy(tabflat.at[fidx_v.at[chunk(k)]],
                              pick_v.at[chunk(k)], psem).wait()
        pltpu.make_async_copy(rowlse_hbm.at[idx_v.at[chunk(k)]],
                              lse_v.at[chunk(k)], lsem).wait()

    def red_body(g, acc):
        sl = pl.ds(pl.multiple_of(g * 16, 16), 16)
        return acc + (lse_v[sl] - pick_v[sl])

    acc = lax.fori_loop(0, g_total, red_body, jnp.zeros((16,), jnp.float32))
    acc_v[...] = acc
    pltpu.sync_copy(acc_v, part_hbm.at[wid])


def _sc_call(n_tokens, table, tabflat, idx_flat, tgt_flat, rowlse):
    tok = n_tokens // NW
    mesh = plsc.VectorSubcoreMesh(core_axis_name="c", subcore_axis_name="s",
                                  num_cores=NC, num_subcores=NS)
    fn = pl.kernel(
        functools.partial(_sc_body, n_tokens),
        out_type=[
            jax.ShapeDtypeStruct((n_tokens, C), jnp.float32),
            jax.ShapeDtypeStruct((NW, 16), jnp.float32),
        ],
        mesh=mesh,
        scratch_types=[
            pltpu.VMEM((tok,), jnp.int32),
            pltpu.VMEM((tok,), jnp.int32),
            pltpu.VMEM((tok,), jnp.int32),
            pltpu.VMEM((tok,), jnp.float32),
            pltpu.VMEM((tok,), jnp.float32),
            pltpu.VMEM((NBUF, 16, C), jnp.float32),
            pltpu.VMEM((16,), jnp.float32),
        ] + [pltpu.SemaphoreType.DMA] * (2 + 2 * NBUF),
        compiler_params=pltpu.CompilerParams(use_tc_tiling_on_sc=False),
    )
    return fn(table, tabflat, idx_flat, tgt_flat, rowlse)


def _loss_body(n_tokens, part_ref, out_ref):
    out_ref[0, 0] = jnp.sum(part_ref[...]) / n_tokens


def _loss(partials, n_tokens):
    out = pl.pallas_call(
        functools.partial(_loss_body, n_tokens),
        out_shape=jax.ShapeDtypeStruct((1, 1), jnp.float32),
        out_specs=pl.BlockSpec(memory_space=pltpu.SMEM),
    )(partials)
    return out[0, 0]


def kernel(idx, targets, token_embedding):
    B, T = idx.shape
    n = B * T
    idx_flat = idx.reshape(n).astype(jnp.int32)
    tgt_flat = targets.reshape(n).astype(jnp.int32)
    rowlse, tabflat = _rowlse(token_embedding)
    logits_flat, partials = _sc_call(n, token_embedding, tabflat, idx_flat,
                                     tgt_flat, rowlse)
    loss = _loss(partials, n)
    return logits_flat.reshape(B, T, C), loss


# speedup vs baseline: 2.1322x; 1.5659x over previous
import functools

import jax
import jax.numpy as jnp
from jax import lax
from jax.experimental import pallas as pl
from jax.experimental.pallas import tpu as pltpu
from jax.experimental.pallas import tpu_sc as plsc

V = 1000
C = 1000
CM = 896
CT = C - CM
NC, NS = 2, 16
NW = NC * NS
NBUF = 4
DIST = 2


def _prep_body(tab_ref, lse_ref, main_ref, tail_ref, copy_ref):
    x = tab_ref[...]
    m = jnp.max(x, axis=1, keepdims=True)
    s = jnp.sum(jnp.exp(x - m), axis=1, keepdims=True)
    lse_ref[...] = jnp.broadcast_to(m + jnp.log(s), (V, 128))
    main_ref[...] = x[:, :CM]
    tail_ref[...] = x[:, CM:]
    copy_ref[...] = x


def _prep(table):
    lse, main, tail, copy = pl.pallas_call(
        _prep_body,
        out_shape=[jax.ShapeDtypeStruct((V, 128), jnp.float32),
                   jax.ShapeDtypeStruct((V, CM), jnp.float32),
                   jax.ShapeDtypeStruct((V, CT), jnp.float32),
                   jax.ShapeDtypeStruct((V, C), jnp.float32)],
    )(table)
    return lse[:, 0] + jnp.zeros((V,), jnp.float32), main, tail, copy


def _main_body(B, tab, aidx, out_hbm, aidx_v, bufs, *sems):
    gsems = sems[:NBUF]
    wsems = sems[NBUF:]
    nb = B // NW
    items = nb * 3
    wid = lax.axis_index("s") * NC + lax.axis_index("c")
    b0 = wid * nb

    pltpu.sync_copy(aidx.at[pl.ds(wid * items * 16, items * 16)], aidx_v)

    def idx16(j):
        return aidx_v[pl.ds(pl.multiple_of(j * 16, 16), 16)]

    def dst(j):
        b = b0 + j // 3
        t0 = pl.multiple_of(lax.rem(j, 3) * 16, 16)
        return out_hbm.at[b, pl.ds(t0, 16), pl.ds(0, CM)]

    for s in range(DIST):
        pltpu.async_copy(tab.at[idx16(s)], bufs.at[s], gsems[s])

    def outer(o, c):
        for s in range(NBUF):
            j = o * NBUF + s
            pltpu.make_async_copy(tab.at[idx16(j)], bufs.at[s],
                                  gsems[s]).wait()
            pltpu.async_copy(bufs.at[s], dst(j), wsems[s])

            s2 = (s + DIST) % NBUF
            jp = j + DIST - NBUF

            @pl.when(jp >= 0)
            def _wait_prev_write():
                pltpu.make_async_copy(bufs.at[s2], dst(jp), wsems[s2]).wait()

            @pl.when(j + DIST < items)
            def _prefetch():
                pltpu.async_copy(tab.at[idx16(j + DIST)], bufs.at[s2],
                                 gsems[s2])
        return c

    lax.fori_loop(0, items // NBUF, outer, 0)

    for j in range(items - NBUF + DIST, items):
        pltpu.make_async_copy(bufs.at[j % NBUF], dst(j),
                              wsems[j % NBUF]).wait()


def _main_call(B, T, tabmain, aidx):
    nb = B // NW
    mesh = plsc.VectorSubcoreMesh(core_axis_name="c", subcore_axis_name="s",
                                  num_cores=NC, num_subcores=NS)
    fn = pl.kernel(
        functools.partial(_main_body, B),
        out_type=jax.ShapeDtypeStruct((B, T, C), jnp.float32),
        mesh=mesh,
        scratch_types=[
            pltpu.VMEM((nb * 48,), jnp.int32),
            pltpu.VMEM((NBUF, 16, CM), jnp.float32),
        ] + [pltpu.SemaphoreType.DMA] * (2 * NBUF),
    )
    return fn(tabmain, aidx)


def _aux_body(n_tokens, nbot, tabtail, tabcopy, tabflat, rowlse_hbm,
              idx_hbm, tgt_hbm, bidx_hbm,
              outtail, outbot, part_hbm,
              idx_v, tgt_v, fidx_v, pick_v, lse_v, bidx_v,
              tbufs, bbufs, acc_v,
              psem, lsem, bgsem, bwsem, *sems):
    gsems = sems[:NBUF]
    wsems = sems[NBUF:]
    tok = n_tokens // NW
    g_total = tok // 16
    wid = lax.axis_index("s") * NC + lax.axis_index("c")
    base = wid * tok

    pltpu.sync_copy(idx_hbm.at[pl.ds(base, tok)], idx_v)
    pltpu.sync_copy(tgt_hbm.at[pl.ds(base, tok)], tgt_v)

    def fidx_body(g, c):
        sl = pl.ds(pl.multiple_of(g * 16, 16), 16)
        fidx_v[sl] = idx_v[sl] * C + tgt_v[sl]
        return c

    lax.fori_loop(0, g_total, fidx_body, 0)

    n_chunks = (tok + 127) // 128
    def chunk(k):
        size = min(128, tok - k * 128)
        return pl.ds(k * 128, size)

    for k in range(n_chunks):
        pltpu.async_copy(tabflat.at[fidx_v.at[chunk(k)]],
                         pick_v.at[chunk(k)], psem)
        pltpu.async_copy(rowlse_hbm.at[idx_v.at[chunk(k)]],
                         lse_v.at[chunk(k)], lsem)

    bper = nbot // NW
    bg = bper // 16
    bbase = wid * bper
    pltpu.sync_copy(bidx_hbm.at[pl.ds(bbase, bper)], bidx_v)

    def bidx16(g):
        return bidx_v[pl.ds(g * 16, 16)]

    def bot_write(g):
        return outbot.at[pl.ds(bbase + g * 16, 16)]

    for g in range(min(2, bg)):
        pltpu.async_copy(tabcopy.at[bidx16(g)], bbufs.at[g % 2], bgsem)
    for g in range(bg):
        pltpu.make_async_copy(tabcopy.at[bidx16(g)], bbufs.at[g % 2],
                              bgsem).wait()
        pltpu.async_copy(bbufs.at[g % 2], bot_write(g), bwsem)
        if g + 2 < bg:
            pltpu.make_async_copy(bbufs.at[g % 2], bot_write(g),
                                  bwsem).wait()
            pltpu.async_copy(tabcopy.at[bidx16(g + 2)], bbufs.at[g % 2],
                             bgsem)

    def idx16(g):
        return idx_v[pl.ds(pl.multiple_of(g * 16, 16), 16)]

    for s in range(DIST):
        pltpu.async_copy(tabtail.at[idx16(s)], tbufs.at[s], gsems[s])

    def outer(o, c):
        for s in range(NBUF):
            g = o * NBUF + s
            pltpu.make_async_copy(tabtail.at[idx16(g)], tbufs.at[s],
                                  gsems[s]).wait()
            row0 = pl.multiple_of(base + g * 16, 16)
            pltpu.async_copy(tbufs.at[s], outtail.at[pl.ds(row0, 16)],
                             wsems[s])

            s2 = (s + DIST) % NBUF
            gp = g + DIST - NBUF

            @pl.when(gp >= 0)
            def _wait_prev_write():
                r0 = pl.multiple_of(base + gp * 16, 16)
                pltpu.make_async_copy(tbufs.at[s2],
                                      outtail.at[pl.ds(r0, 16)],
                                      wsems[s2]).wait()

            @pl.when(g + DIST < g_total)
            def _prefetch():
                pltpu.async_copy(tabtail.at[idx16(g + DIST)], tbufs.at[s2],
                                 gsems[s2])
        return c

    lax.fori_loop(0, g_total // NBUF, outer, 0)

    for g in range(g_total - NBUF + DIST, g_total):
        s = g % NBUF
        row0 = pl.multiple_of(base + g * 16, 16)
        pltpu.make_async_copy(tbufs.at[s], outtail.at[pl.ds(row0, 16)],
                              wsems[s]).wait()

    for g in range(max(0, bg - 2), bg):
        pltpu.make_async_copy(bbufs.at[g % 2],
                              outbot.at[pl.ds(bbase + g * 16, 16)],
                              bwsem).wait()

    for k in range(n_chunks):
        pltpu.make_async_copy(tabflat.at[fidx_v.at[chunk(k)]],
                              pick_v.at[chunk(k)], psem).wait()
        pltpu.make_async_copy(rowlse_hbm.at[idx_v.at[chunk(k)]],
                              lse_v.at[chunk(k)], lsem).wait()

    def red_body(g, acc):
        sl = pl.ds(pl.multiple_of(g * 16, 16), 16)
        return acc + (lse_v[sl] - pick_v[sl])

    acc = lax.fori_loop(0, g_total, red_body, jnp.zeros((16,), jnp.float32))
    acc_v[...] = acc
    pltpu.sync_copy(acc_v, part_hbm.at[wid])


def _aux_call(n_tokens, nbot, tabtail, tabcopy, tabflat, rowlse,
              idx_flat, tgt_flat, bidx):
    tok = n_tokens // NW
    mesh = plsc.VectorSubcoreMesh(core_axis_name="c", subcore_axis_name="s",
                                  num_cores=NC, num_subcores=NS)
    fn = pl.kernel(
        functools.partial(_aux_body, n_tokens, nbot),
        out_type=[
            jax.ShapeDtypeStruct((n_tokens, CT), jnp.float32),
            jax.ShapeDtypeStruct((nbot, C), jnp.float32),
            jax.ShapeDtypeStruct((NW, 16), jnp.float32),
        ],
        mesh=mesh,
        scratch_types=[
            pltpu.VMEM((tok,), jnp.int32),
            pltpu.VMEM((tok,), jnp.int32),
            pltpu.VMEM((tok,), jnp.int32),
            pltpu.VMEM((tok,), jnp.float32),
            pltpu.VMEM((tok,), jnp.float32),
            pltpu.VMEM((nbot // NW,), jnp.int32),
            pltpu.VMEM((NBUF, 16, CT), jnp.float32),
            pltpu.VMEM((2, 16, C), jnp.float32),
            pltpu.VMEM((16,), jnp.float32),
        ] + [pltpu.SemaphoreType.DMA] * (4 + 2 * NBUF),
        compiler_params=pltpu.CompilerParams(use_tc_tiling_on_sc=False),
    )
    return fn(tabtail, tabcopy, tabflat, rowlse, idx_flat, tgt_flat, bidx)


def _loss_body(n_tokens, part_ref, out_ref):
    out_ref[0, 0] = jnp.sum(part_ref[...]) / n_tokens


def _loss(partials, n_tokens):
    out = pl.pallas_call(
        functools.partial(_loss_body, n_tokens),
        out_shape=jax.ShapeDtypeStruct((1, 1), jnp.float32),
        out_specs=pl.BlockSpec(memory_space=pltpu.SMEM),
    )(partials)
    return out[0, 0]


def kernel(idx, targets, token_embedding):
    B, T = idx.shape
    n = B * T
    idx32 = idx.astype(jnp.int32)
    idx_flat = idx32.reshape(n)
    tgt_flat = targets.reshape(n).astype(jnp.int32)
    aidx = idx32[:, :48].reshape(B * 48)
    bidx = idx32[:, 48:].reshape(B * 2)
    tabflat = token_embedding.reshape(V * C)

    rowlse, tabmain, tabtail, tabcopy = _prep(token_embedding)
    out3d = _main_call(B, T, tabmain, aidx)
    outtail, outbot, partials = _aux_call(n, B * 2, tabtail, tabcopy, tabflat,
                                          rowlse, idx_flat, tgt_flat, bidx)
    logits = lax.dynamic_update_slice(out3d, outtail.reshape(B, T, CT),
                                      (0, 0, CM))
    logits = lax.dynamic_update_slice(logits, outbot.reshape(B, 2, C),
                                      (0, 48, 0))
    loss = _loss(partials, n)
    return logits, loss


# speedup vs baseline: 2.4280x; 1.1387x over previous
import functools

import jax
import jax.numpy as jnp
from jax import lax
from jax.experimental import pallas as pl
from jax.experimental.pallas import tpu as pltpu
from jax.experimental.pallas import tpu_sc as plsc

V = 1000
C = 1000
CM = 896
CT = C - CM
NC, NS = 2, 16
NW = NC * NS
NBUF = 4
DIST = 2


def _prep_body(tab_ref, lse_ref, main_ref, thi_ref, tlo_ref,
               hi_ref, lo_ref):
    x = tab_ref[...]
    m = jnp.max(x, axis=1, keepdims=True)
    s = jnp.sum(jnp.exp(x - m), axis=1, keepdims=True)
    lse_ref[...] = jnp.broadcast_to(m + jnp.log(s), (V, 128))
    main_ref[...] = x[:, :CM]
    hi = x.astype(jnp.bfloat16)
    lo = (x - hi.astype(jnp.float32)).astype(jnp.bfloat16)
    thi_ref[...] = hi[:, CM:]
    tlo_ref[...] = lo[:, CM:]
    hi_ref[...] = hi
    lo_ref[...] = lo


def _prep(table):
    lse, main, thi, tlo, hi, lo = pl.pallas_call(
        _prep_body,
        out_shape=[jax.ShapeDtypeStruct((V, 128), jnp.float32),
                   jax.ShapeDtypeStruct((V, CM), jnp.float32),
                   jax.ShapeDtypeStruct((V, CT), jnp.bfloat16),
                   jax.ShapeDtypeStruct((V, CT), jnp.bfloat16),
                   jax.ShapeDtypeStruct((V, C), jnp.bfloat16),
                   jax.ShapeDtypeStruct((V, C), jnp.bfloat16)],
    )(table)
    return lse[:, 0] + jnp.zeros((V,), jnp.float32), main, thi, tlo, hi, lo


_DN0 = (((0,), (0,)), ((), ()))


def _onehot_mm_body(nsel, idx_ref, hi_ref, lo_ref, out_ref):
    m = idx_ref.shape[1]
    bc = jnp.broadcast_to(idx_ref[...], (V, m))
    ohT = (bc == lax.broadcasted_iota(jnp.int32, (V, m), 0)
           ).astype(jnp.bfloat16)
    acc = lax.dot_general(ohT, hi_ref[...], _DN0,
                          preferred_element_type=jnp.float32)
    acc = acc + lax.dot_general(ohT, lo_ref[...], _DN0,
                                preferred_element_type=jnp.float32)
    out_ref[...] = acc


def _onehot_mm(idx_row, hi, lo, nsel, mblk):
    n = idx_row.shape[1]
    return pl.pallas_call(
        functools.partial(_onehot_mm_body, nsel),
        grid=(n // mblk,),
        in_specs=[pl.BlockSpec((1, mblk), lambda i: (0, i)),
                  pl.BlockSpec((V, nsel), lambda i: (0, 0)),
                  pl.BlockSpec((V, nsel), lambda i: (0, 0))],
        out_specs=pl.BlockSpec((mblk, nsel), lambda i: (i, 0)),
        out_shape=jax.ShapeDtypeStruct((n, nsel), jnp.float32),
    )(idx_row, hi, lo)


def _main_body(B, tab, aidx, out_hbm, aidx_v, bufs, *sems):
    gsems = sems[:NBUF]
    wsems = sems[NBUF:]
    nb = B // NW
    items = nb * 3
    wid = lax.axis_index("s") * NC + lax.axis_index("c")
    b0 = wid * nb

    pltpu.sync_copy(aidx.at[pl.ds(wid * items * 16, items * 16)], aidx_v)

    def idx16(j):
        return aidx_v[pl.ds(pl.multiple_of(j * 16, 16), 16)]

    def dst(j):
        b = b0 + j // 3
        t0 = pl.multiple_of(lax.rem(j, 3) * 16, 16)
        return out_hbm.at[b, pl.ds(t0, 16), pl.ds(0, CM)]

    for s in range(DIST):
        pltpu.async_copy(tab.at[idx16(s)], bufs.at[s], gsems[s])

    def outer(o, c):
        for s in range(NBUF):
            j = o * NBUF + s
            pltpu.make_async_copy(tab.at[idx16(j)], bufs.at[s],
                                  gsems[s]).wait()
            pltpu.async_copy(bufs.at[s], dst(j), wsems[s])

            s2 = (s + DIST) % NBUF
            jp = j + DIST - NBUF

            @pl.when(jp >= 0)
            def _wait_prev_write():
                pltpu.make_async_copy(bufs.at[s2], dst(jp), wsems[s2]).wait()

            @pl.when(j + DIST < items)
            def _prefetch():
                pltpu.async_copy(tab.at[idx16(j + DIST)], bufs.at[s2],
                                 gsems[s2])
        return c

    lax.fori_loop(0, items // NBUF, outer, 0)

    for j in range(items - NBUF + DIST, items):
        pltpu.make_async_copy(bufs.at[j % NBUF], dst(j),
                              wsems[j % NBUF]).wait()


def _main_call(B, T, tabmain, aidx):
    nb = B // NW
    mesh = plsc.VectorSubcoreMesh(core_axis_name="c", subcore_axis_name="s",
                                  num_cores=NC, num_subcores=NS)
    fn = pl.kernel(
        functools.partial(_main_body, B),
        out_type=jax.ShapeDtypeStruct((B, T, C), jnp.float32),
        mesh=mesh,
        scratch_types=[
            pltpu.VMEM((nb * 48,), jnp.int32),
            pltpu.VMEM((NBUF, 16, CM), jnp.float32),
        ] + [pltpu.SemaphoreType.DMA] * (2 * NBUF),
    )
    return fn(tabmain, aidx)


def _aux_body(n_tokens, tabflat, rowlse_hbm, idx_hbm, tgt_hbm, part_hbm,
              idx_v, tgt_v, fidx_v, pick_v, lse_v, acc_v, psem, lsem):
    tok = n_tokens // NW
    g_total = tok // 16
    wid = lax.axis_index("s") * NC + lax.axis_index("c")
    base = wid * tok

    pltpu.sync_copy(idx_hbm.at[pl.ds(base, tok)], idx_v)
    pltpu.sync_copy(tgt_hbm.at[pl.ds(base, tok)], tgt_v)

    def fidx_body(g, c):
        sl = pl.ds(pl.multiple_of(g * 16, 16), 16)
        fidx_v[sl] = idx_v[sl] * C + tgt_v[sl]
        return c

    lax.fori_loop(0, g_total, fidx_body, 0)

    n_chunks = (tok + 127) // 128
    def chunk(k):
        size = min(128, tok - k * 128)
        return pl.ds(k * 128, size)

    for k in range(n_chunks):
        pltpu.async_copy(tabflat.at[fidx_v.at[chunk(k)]],
                         pick_v.at[chunk(k)], psem)
        pltpu.async_copy(rowlse_hbm.at[idx_v.at[chunk(k)]],
                         lse_v.at[chunk(k)], lsem)
    for k in range(n_chunks):
        pltpu.make_async_copy(tabflat.at[fidx_v.at[chunk(k)]],
                              pick_v.at[chunk(k)], psem).wait()
        pltpu.make_async_copy(rowlse_hbm.at[idx_v.at[chunk(k)]],
                              lse_v.at[chunk(k)], lsem).wait()

    def red_body(g, acc):
        sl = pl.ds(pl.multiple_of(g * 16, 16), 16)
        return acc + (lse_v[sl] - pick_v[sl])

    acc = lax.fori_loop(0, g_total, red_body, jnp.zeros((16,), jnp.float32))
    acc_v[...] = acc
    pltpu.sync_copy(acc_v, part_hbm.at[wid])


def _aux_call(n_tokens, tabflat, rowlse, idx_flat, tgt_flat):
    tok = n_tokens // NW
    mesh = plsc.VectorSubcoreMesh(core_axis_name="c", subcore_axis_name="s",
                                  num_cores=NC, num_subcores=NS)
    fn = pl.kernel(
        functools.partial(_aux_body, n_tokens),
        out_type=jax.ShapeDtypeStruct((NW, 16), jnp.float32),
        mesh=mesh,
        scratch_types=[
            pltpu.VMEM((tok,), jnp.int32),
            pltpu.VMEM((tok,), jnp.int32),
            pltpu.VMEM((tok,), jnp.int32),
            pltpu.VMEM((tok,), jnp.float32),
            pltpu.VMEM((tok,), jnp.float32),
            pltpu.VMEM((16,), jnp.float32),
        ] + [pltpu.SemaphoreType.DMA] * 2,
        compiler_params=pltpu.CompilerParams(use_tc_tiling_on_sc=False),
    )
    return fn(tabflat, rowlse, idx_flat, tgt_flat)


def _loss_body(n_tokens, part_ref, out_ref):
    out_ref[0, 0] = jnp.sum(part_ref[...]) / n_tokens


def _loss(partials, n_tokens):
    out = pl.pallas_call(
        functools.partial(_loss_body, n_tokens),
        out_shape=jax.ShapeDtypeStruct((1, 1), jnp.float32),
        out_specs=pl.BlockSpec(memory_space=pltpu.SMEM),
    )(partials)
    return out[0, 0]


def kernel(idx, targets, token_embedding):
    B, T = idx.shape
    n = B * T
    idx32 = idx.astype(jnp.int32)
    idx_flat = idx32.reshape(n)
    tgt_flat = targets.reshape(n).astype(jnp.int32)
    aidx = idx32[:, :48].reshape(B * 48)
    bidx2d = idx32[:, 48:]
    tabflat = token_embedding.reshape(V * C)

    rowlse, tabmain, thi, tlo, hi, lo = _prep(token_embedding)
    out3d = _main_call(B, T, tabmain, aidx)
    tail2d = _onehot_mm(idx_flat.reshape(1, n), thi, tlo, CT, 512)
    bot2d = _onehot_mm(bidx2d.reshape(1, 2 * B), hi, lo, C, 2 * B)
    partials = _aux_call(n, tabflat, rowlse, idx_flat, tgt_flat)
    logits = lax.dynamic_update_slice(out3d, tail2d.reshape(B, T, CT),
                                      (0, 0, CM))
    logits = lax.dynamic_update_slice(logits, bot2d.reshape(B, 2, C),
                                      (0, 48, 0))
    loss = _loss(partials, n)
    return logits, loss


# speedup vs baseline: 2.5054x; 1.0319x over previous
import functools

import jax
import jax.numpy as jnp
from jax import lax
from jax.experimental import pallas as pl
from jax.experimental.pallas import tpu as pltpu
from jax.experimental.pallas import tpu_sc as plsc

V = 1000
C = 1000
CM = 896
CT = C - CM
NC, NS = 2, 16
NW = NC * NS
NBUF = 6
DIST = 3


def _prep_body(tab_ref, lse_ref, main_ref, thi_ref, tlo_ref,
               hi_ref, lo_ref):
    x = tab_ref[...]
    m = jnp.max(x, axis=1, keepdims=True)
    s = jnp.sum(jnp.exp(x - m), axis=1, keepdims=True)
    lse_ref[...] = jnp.broadcast_to(m + jnp.log(s), (V, 128))
    main_ref[...] = x[:, :CM]
    hi = x.astype(jnp.bfloat16)
    lo = (x - hi.astype(jnp.float32)).astype(jnp.bfloat16)
    thi_ref[...] = hi[:, CM:]
    tlo_ref[...] = lo[:, CM:]
    hi_ref[...] = hi
    lo_ref[...] = lo


def _prep(table):
    lse, main, thi, tlo, hi, lo = pl.pallas_call(
        _prep_body,
        out_shape=[jax.ShapeDtypeStruct((V, 128), jnp.float32),
                   jax.ShapeDtypeStruct((V, CM), jnp.float32),
                   jax.ShapeDtypeStruct((V, CT), jnp.bfloat16),
                   jax.ShapeDtypeStruct((V, CT), jnp.bfloat16),
                   jax.ShapeDtypeStruct((V, C), jnp.bfloat16),
                   jax.ShapeDtypeStruct((V, C), jnp.bfloat16)],
    )(table)
    return lse, main, thi, tlo, hi, lo


_DN0 = (((0,), (0,)), ((), ()))


def _onehot_mm_body(nsel, count, idx_ref, hi_ref, lo_ref, out_ref, *cnt_ref):
    m = idx_ref.shape[1]
    bc = jnp.broadcast_to(idx_ref[...], (V, m))
    ohT = (bc == lax.broadcasted_iota(jnp.int32, (V, m), 0)
           ).astype(jnp.bfloat16)
    acc = lax.dot_general(ohT, hi_ref[...], _DN0,
                          preferred_element_type=jnp.float32)
    acc = acc + lax.dot_general(ohT, lo_ref[...], _DN0,
                                preferred_element_type=jnp.float32)
    out_ref[...] = acc
    if count:
        c = jnp.sum(ohT.astype(jnp.float32), axis=1, keepdims=True)
        blk = jnp.broadcast_to(c, (V, 128))

        @pl.when(pl.program_id(0) == 0)
        def _init():
            cnt_ref[0][...] = blk

        @pl.when(pl.program_id(0) != 0)
        def _acc():
            cnt_ref[0][...] += blk


def _onehot_mm(idx_row, hi, lo, nsel, mblk, count=False):
    n = idx_row.shape[1]
    out_shape = [jax.ShapeDtypeStruct((n, nsel), jnp.float32)]
    out_specs = [pl.BlockSpec((mblk, nsel), lambda i: (i, 0))]
    if count:
        out_shape.append(jax.ShapeDtypeStruct((V, 128), jnp.float32))
        out_specs.append(pl.BlockSpec((V, 128), lambda i: (0, 0)))
    res = pl.pallas_call(
        functools.partial(_onehot_mm_body, nsel, count),
        grid=(n // mblk,),
        in_specs=[pl.BlockSpec((1, mblk), lambda i: (0, i)),
                  pl.BlockSpec((V, nsel), lambda i: (0, 0)),
                  pl.BlockSpec((V, nsel), lambda i: (0, 0))],
        out_specs=out_specs,
        out_shape=out_shape,
    )(idx_row, hi, lo)
    return res if count else res[0]


def _main_body(B, tab, aidx, out_hbm, aidx_v, bufs, *sems):
    gsems = sems[:NBUF]
    wsems = sems[NBUF:]
    nb = B // NW
    items = nb * 3
    wid = lax.axis_index("s") * NC + lax.axis_index("c")
    b0 = wid * nb

    pltpu.sync_copy(aidx.at[pl.ds(wid * items * 16, items * 16)], aidx_v)

    def idx16(j):
        return aidx_v[pl.ds(pl.multiple_of(j * 16, 16), 16)]

    def dst(j):
        b = b0 + j // 3
        t0 = pl.multiple_of(lax.rem(j, 3) * 16, 16)
        return out_hbm.at[b, pl.ds(t0, 16), pl.ds(0, CM)]

    for s in range(DIST):
        pltpu.async_copy(tab.at[idx16(s)], bufs.at[s], gsems[s])

    def outer(o, c):
        for s in range(NBUF):
            j = o * NBUF + s
            pltpu.make_async_copy(tab.at[idx16(j)], bufs.at[s],
                                  gsems[s]).wait()
            pltpu.async_copy(bufs.at[s], dst(j), wsems[s])

            s2 = (s + DIST) % NBUF
            jp = j + DIST - NBUF

            @pl.when(jp >= 0)
            def _wait_prev_write():
                pltpu.make_async_copy(bufs.at[s2], dst(jp), wsems[s2]).wait()

            @pl.when(j + DIST < items)
            def _prefetch():
                pltpu.async_copy(tab.at[idx16(j + DIST)], bufs.at[s2],
                                 gsems[s2])
        return c

    lax.fori_loop(0, items // NBUF, outer, 0)

    for j in range(items - NBUF + DIST, items):
        pltpu.make_async_copy(bufs.at[j % NBUF], dst(j),
                              wsems[j % NBUF]).wait()


def _main_call(B, T, tabmain, aidx):
    nb = B // NW
    mesh = plsc.VectorSubcoreMesh(core_axis_name="c", subcore_axis_name="s",
                                  num_cores=NC, num_subcores=NS)
    fn = pl.kernel(
        functools.partial(_main_body, B),
        out_type=jax.ShapeDtypeStruct((B, T, C), jnp.float32),
        mesh=mesh,
        scratch_types=[
            pltpu.VMEM((nb * 48,), jnp.int32),
            pltpu.VMEM((NBUF, 16, CM), jnp.float32),
        ] + [pltpu.SemaphoreType.DMA] * (2 * NBUF),
    )
    return fn(tabmain, aidx)


def _aux_body(n_tokens, tabflat, idx_hbm, tgt_hbm, part_hbm,
              idx_v, tgt_v, fidx_v, pick_v, acc_v, psem):
    tok = n_tokens // NW
    g_total = tok // 16
    wid = lax.axis_index("s") * NC + lax.axis_index("c")
    base = wid * tok

    pltpu.sync_copy(idx_hbm.at[pl.ds(base, tok)], idx_v)
    pltpu.sync_copy(tgt_hbm.at[pl.ds(base, tok)], tgt_v)

    def fidx_body(g, c):
        sl = pl.ds(pl.multiple_of(g * 16, 16), 16)
        fidx_v[sl] = idx_v[sl] * C + tgt_v[sl]
        return c

    lax.fori_loop(0, g_total, fidx_body, 0)

    n_chunks = (tok + 127) // 128
    def chunk(k):
        size = min(128, tok - k * 128)
        return pl.ds(k * 128, size)

    for k in range(n_chunks):
        pltpu.async_copy(tabflat.at[fidx_v.at[chunk(k)]],
                         pick_v.at[chunk(k)], psem)
    for k in range(n_chunks):
        pltpu.make_async_copy(tabflat.at[fidx_v.at[chunk(k)]],
                              pick_v.at[chunk(k)], psem).wait()

    def red_body(g, acc):
        sl = pl.ds(pl.multiple_of(g * 16, 16), 16)
        return acc + pick_v[sl]

    acc = lax.fori_loop(0, g_total, red_body, jnp.zeros((16,), jnp.float32))
    acc_v[...] = acc
    pltpu.sync_copy(acc_v, part_hbm.at[wid])


def _aux_call(n_tokens, tabflat, idx_flat, tgt_flat):
    tok = n_tokens // NW
    mesh = plsc.VectorSubcoreMesh(core_axis_name="c", subcore_axis_name="s",
                                  num_cores=NC, num_subcores=NS)
    fn = pl.kernel(
        functools.partial(_aux_body, n_tokens),
        out_type=jax.ShapeDtypeStruct((NW, 16), jnp.float32),
        mesh=mesh,
        scratch_types=[
            pltpu.VMEM((tok,), jnp.int32),
            pltpu.VMEM((tok,), jnp.int32),
            pltpu.VMEM((tok,), jnp.int32),
            pltpu.VMEM((tok,), jnp.float32),
            pltpu.VMEM((16,), jnp.float32),
        ] + [pltpu.SemaphoreType.DMA] * 1,
        compiler_params=pltpu.CompilerParams(use_tc_tiling_on_sc=False),
    )
    return fn(tabflat, idx_flat, tgt_flat)


def _loss_body(n_tokens, part_ref, cnt_ref, lse_ref, out_ref):
    lse_sum = jnp.sum(cnt_ref[:, :1] * lse_ref[:, :1])
    out_ref[0, 0] = (lse_sum - jnp.sum(part_ref[...])) / n_tokens


def _loss(partials, counts, lse2d, n_tokens):
    out = pl.pallas_call(
        functools.partial(_loss_body, n_tokens),
        out_shape=jax.ShapeDtypeStruct((1, 1), jnp.float32),
        out_specs=pl.BlockSpec(memory_space=pltpu.SMEM),
    )(partials, counts, lse2d)
    return out[0, 0]


def kernel(idx, targets, token_embedding):
    B, T = idx.shape
    n = B * T
    idx32 = idx.astype(jnp.int32)
    idx_flat = idx32.reshape(n)
    tgt_flat = targets.reshape(n).astype(jnp.int32)
    aidx = idx32[:, :48].reshape(B * 48)
    bidx2d = idx32[:, 48:]
    tabflat = token_embedding.reshape(V * C)

    lse2d, tabmain, thi, tlo, hi, lo = _prep(token_embedding)
    partials = _aux_call(n, tabflat, idx_flat, tgt_flat)
    tail2d, counts = _onehot_mm(idx_flat.reshape(1, n), thi, tlo, CT, 512,
                                count=True)
    bot2d = _onehot_mm(bidx2d.reshape(1, 2 * B), hi, lo, C, 2 * B)
    out3d = _main_call(B, T, tabmain, aidx)
    logits = lax.dynamic_update_slice(out3d, tail2d.reshape(B, T, CT),
                                      (0, 0, CM))
    logits = lax.dynamic_update_slice(logits, bot2d.reshape(B, 2, C),
                                      (0, 48, 0))
    loss = _loss(partials, counts, lse2d, n)
    return logits, loss
